# baseline (speedup 1.0000x reference)
import math

import jax
import jax.numpy as jnp
from jax.experimental import pallas as pl
from jax.experimental.pallas import tpu as pltpu

_NUM_FEATURES = 128
_NUM_PIDS = 100000
_NUM_CQ = 5000
_OIM_SCALAR = 30.0
_B = 128
_BLK = 20000
_NLUT = _NUM_PIDS // _BLK
_NBLK = _NLUT
_LOG2E = math.log2(math.e)
_LN2 = math.log(2.0)


_CHUNK = 4000
_NCHUNK = _BLK // _CHUNK


def _oim_kernel(lab_ref, inputs_ref, cls_ref, lut_ref, cq_ref, out_ref,
                m_ref, s_ref, picked_ref, safe_ref, valid_ref, x_ref,
                xl_ref):
    i = pl.program_id(0)

    @pl.when(i == 0)
    def _init():
        t_r = lab_ref[...] - 1
        row = jax.lax.broadcasted_iota(jnp.int32, (_B, _B), 0)
        col = jax.lax.broadcasted_iota(jnp.int32, (_B, _B), 1)
        diag = row == col
        t_mat = jnp.broadcast_to(t_r, (_B, _B))
        t_c = jnp.sum(jnp.where(diag, t_mat, 0), axis=0, keepdims=True)
        t_cmat = jnp.broadcast_to(t_c, (_B, _B))
        eq = t_mat == t_cmat
        earlier = col < row
        mask_r = t_r >= _NUM_PIDS
        any_earlier = jnp.sum((eq & earlier).astype(jnp.int32), axis=1,
                              keepdims=True) > 0
        first_r = mask_r & jnp.logical_not(any_earlier)
        first_c = jnp.sum(jnp.where(diag & jnp.broadcast_to(first_r, (_B, _B)),
                                    1, 0), axis=0, keepdims=True) > 0
        less = t_cmat < t_mat
        rank = jnp.sum((jnp.broadcast_to(first_c, (_B, _B)) & less)
                       .astype(jnp.int32), axis=1, keepdims=True)
        label = jnp.where(mask_r, _NUM_PIDS + rank % _NUM_CQ, t_r)
        valid = label != -1
        safe_ref[...] = jnp.where(valid, label, 0)
        valid_ref[...] = valid.astype(jnp.float32)
        m_ref[...] = jnp.full((_B, 1), -1e30, jnp.float32)
        s_ref[...] = jnp.zeros((_B, 1), jnp.float32)
        picked_ref[...] = jnp.zeros((_B, 1), jnp.float32)
        xf = inputs_ref[...] * (cls_ref[...] * (_OIM_SCALAR * _LOG2E))
        xh = xf.astype(jnp.bfloat16)
        x_ref[...] = xh
        xl_ref[...] = (xf - xh.astype(jnp.float32)).astype(jnp.bfloat16)

    def _sweep(logits, base):
        cols = jax.lax.broadcasted_iota(jnp.int32, logits.shape, 1)
        bm = jnp.max(logits, axis=1, keepdims=True)
        sel = cols == safe_ref[...] - base
        picked_ref[...] += jnp.sum(jnp.where(sel, logits, 0.0), axis=1,
                                   keepdims=True)
        sc = jnp.sum(jnp.exp2(logits - bm), axis=1, keepdims=True)
        return bm, sc

    xh = x_ref[...]
    xl = xl_ref[...]
    _dims = (((1,), (1,)), ((), ()))

    def _mm3(table):
        th = table.astype(jnp.bfloat16)
        tl = (table - th.astype(jnp.float32)).astype(jnp.bfloat16)
        hh = jax.lax.dot_general(xh, th, _dims,
                                 preferred_element_type=jnp.float32)
        hl = jax.lax.dot_general(xh, tl, _dims,
                                 preferred_element_type=jnp.float32)
        lh = jax.lax.dot_general(xl, th, _dims,
                                 preferred_element_type=jnp.float32)
        return hh + (hl + lh)

    parts = []
    for c in range(_NCHUNK):
        logits = _mm3(lut_ref[pl.ds(c * _CHUNK, _CHUNK), :])
        parts.append(_sweep(logits, i * _BLK + c * _CHUNK))

    m_old = m_ref[...]
    m_new = m_old
    for bm, _ in parts:
        m_new = jnp.maximum(m_new, bm)
    s_new = s_ref[...] * jnp.exp2(m_old - m_new)
    for bm, sc in parts:
        s_new = s_new + sc * jnp.exp2(bm - m_new)
    s_ref[...] = s_new
    m_ref[...] = m_new

    @pl.when(i == _NBLK - 1)
    def _final():
        cq_logits = _mm3(cq_ref[...])
        bm, sc = _sweep(cq_logits, _NUM_PIDS)
        m2 = jnp.maximum(m_ref[...], bm)
        s2 = (s_ref[...] * jnp.exp2(m_ref[...] - m2)
              + sc * jnp.exp2(bm - m2))
        lse2 = m2 + jnp.log2(s2)
        nll = (lse2 - picked_ref[...]) * _LN2
        valid = valid_ref[...]
        cnt = jnp.sum(valid, axis=0, keepdims=True)
        total = jnp.sum(nll * valid, axis=0, keepdims=True)
        out_ref[...] = total / jnp.maximum(cnt, 1.0)


def kernel(inputs, roi_label, cls_scores, images, proposals, GT_info, lut, cq):
    del images, proposals, GT_info
    lab = roi_label.reshape(_B, 1).astype(jnp.int32)
    out = pl.pallas_call(
        _oim_kernel,
        grid=(_NBLK,),
        in_specs=[
            pl.BlockSpec((_B, 1), lambda i: (0, 0)),
            pl.BlockSpec((_B, _NUM_FEATURES), lambda i: (0, 0)),
            pl.BlockSpec((_B, 1), lambda i: (0, 0)),
            pl.BlockSpec((_BLK, _NUM_FEATURES), lambda i: (i, 0)),
            pl.BlockSpec((_NUM_CQ, _NUM_FEATURES), lambda i: (0, 0)),
        ],
        out_specs=pl.BlockSpec((1, 1), lambda i: (0, 0)),
        out_shape=jax.ShapeDtypeStruct((1, 1), jnp.float32),
        scratch_shapes=[
            pltpu.VMEM((_B, 1), jnp.float32),
            pltpu.VMEM((_B, 1), jnp.float32),
            pltpu.VMEM((_B, 1), jnp.float32),
            pltpu.VMEM((_B, 1), jnp.int32),
            pltpu.VMEM((_B, 1), jnp.float32),
            pltpu.VMEM((_B, _NUM_FEATURES), jnp.bfloat16),
            pltpu.VMEM((_B, _NUM_FEATURES), jnp.bfloat16),
        ],
        compiler_params=pltpu.CompilerParams(
            dimension_semantics=("arbitrary",)),
    )(lab, inputs, cls_scores, lut, cq)
    return out[0, 0]


# speedup vs baseline: 1.4520x; 1.4520x over previous
import math

import jax
import jax.numpy as jnp
from jax.experimental import pallas as pl
from jax.experimental.pallas import tpu as pltpu

_NUM_FEATURES = 128
_NUM_PIDS = 100000
_NUM_CQ = 5000
_OIM_SCALAR = 30.0
_B = 128
_BLK = 20000
_NLUT = _NUM_PIDS // _BLK
_NBLK = _NLUT
_LOG2E = math.log2(math.e)
_LN2 = math.log(2.0)


_CHUNK = 4000
_NCHUNK = _BLK // _CHUNK


def _oim_kernel(lab_ref, inputs_ref, cls_ref, lut_ref, cq_ref, out_ref,
                m_ref, s_ref, picked_ref, safe_ref, valid_ref, x_ref):
    i = pl.program_id(0)

    @pl.when(i == 0)
    def _init():
        t_r = lab_ref[...] - 1
        row = jax.lax.broadcasted_iota(jnp.int32, (_B, _B), 0)
        col = jax.lax.broadcasted_iota(jnp.int32, (_B, _B), 1)
        diag = row == col
        t_mat = jnp.broadcast_to(t_r, (_B, _B))
        t_c = jnp.sum(jnp.where(diag, t_mat, 0), axis=0, keepdims=True)
        t_cmat = jnp.broadcast_to(t_c, (_B, _B))
        eq = t_mat == t_cmat
        earlier = col < row
        mask_r = t_r >= _NUM_PIDS
        any_earlier = jnp.sum((eq & earlier).astype(jnp.int32), axis=1,
                              keepdims=True) > 0
        first_r = mask_r & jnp.logical_not(any_earlier)
        first_c = jnp.sum(jnp.where(diag & jnp.broadcast_to(first_r, (_B, _B)),
                                    1, 0), axis=0, keepdims=True) > 0
        less = t_cmat < t_mat
        rank = jnp.sum((jnp.broadcast_to(first_c, (_B, _B)) & less)
                       .astype(jnp.int32), axis=1, keepdims=True)
        label = jnp.where(mask_r, _NUM_PIDS + rank % _NUM_CQ, t_r)
        valid = label != -1
        safe_ref[...] = jnp.where(valid, label, 0)
        valid_ref[...] = valid.astype(jnp.float32)
        m_ref[...] = jnp.full((_B, 1), -1e30, jnp.float32)
        s_ref[...] = jnp.zeros((_B, 1), jnp.float32)
        picked_ref[...] = jnp.zeros((_B, 1), jnp.float32)
        x_ref[...] = inputs_ref[...] * (cls_ref[...] * (_OIM_SCALAR * _LOG2E))

    def _sweep(logits, base):
        cols = jax.lax.broadcasted_iota(jnp.int32, logits.shape, 1)
        bm = jnp.max(logits, axis=1, keepdims=True)
        sel = cols == safe_ref[...] - base
        pk = jnp.sum(jnp.where(sel, logits, 0.0), axis=1, keepdims=True)
        sc = jnp.sum(jnp.exp2(logits - bm), axis=1, keepdims=True)
        return bm, sc, pk

    x = x_ref[...]
    parts = []
    for c in range(_NCHUNK):
        logits = jax.lax.dot_general(
            x, lut_ref[pl.ds(c * _CHUNK, _CHUNK), :], (((1,), (1,)), ((), ())),
            preferred_element_type=jnp.float32)
        parts.append(_sweep(logits, i * _BLK + c * _CHUNK))

    m_old = m_ref[...]
    m_new = m_old
    for bm, _, _ in parts:
        m_new = jnp.maximum(m_new, bm)
    s_new = s_ref[...] * jnp.exp2(m_old - m_new)
    pk_new = picked_ref[...]
    for bm, sc, pk in parts:
        s_new = s_new + sc * jnp.exp2(bm - m_new)
        pk_new = pk_new + pk
    s_ref[...] = s_new
    m_ref[...] = m_new
    picked_ref[...] = pk_new

    @pl.when(i == _NBLK - 1)
    def _final():
        cq_logits = jax.lax.dot_general(
            x, cq_ref[...], (((1,), (1,)), ((), ())),
            preferred_element_type=jnp.float32)
        bm, sc, pk = _sweep(cq_logits, _NUM_PIDS)
        picked_ref[...] += pk
        m2 = jnp.maximum(m_ref[...], bm)
        s2 = (s_ref[...] * jnp.exp2(m_ref[...] - m2)
              + sc * jnp.exp2(bm - m2))
        lse2 = m2 + jnp.log2(s2)
        nll = (lse2 - picked_ref[...]) * _LN2
        valid = valid_ref[...]
        cnt = jnp.sum(valid, axis=0, keepdims=True)
        total = jnp.sum(nll * valid, axis=0, keepdims=True)
        out_ref[...] = total / jnp.maximum(cnt, 1.0)


def kernel(inputs, roi_label, cls_scores, images, proposals, GT_info, lut, cq):
    del images, proposals, GT_info
    lab = roi_label.reshape(_B, 1).astype(jnp.int32)
    out = pl.pallas_call(
        _oim_kernel,
        grid=(_NBLK,),
        in_specs=[
            pl.BlockSpec((_B, 1), lambda i: (0, 0)),
            pl.BlockSpec((_B, _NUM_FEATURES), lambda i: (0, 0)),
            pl.BlockSpec((_B, 1), lambda i: (0, 0)),
            pl.BlockSpec((_BLK, _NUM_FEATURES), lambda i: (i, 0)),
            pl.BlockSpec((_NUM_CQ, _NUM_FEATURES), lambda i: (0, 0)),
        ],
        out_specs=pl.BlockSpec((1, 1), lambda i: (0, 0)),
        out_shape=jax.ShapeDtypeStruct((1, 1), jnp.float32),
        scratch_shapes=[
            pltpu.VMEM((_B, 1), jnp.float32),
            pltpu.VMEM((_B, 1), jnp.float32),
            pltpu.VMEM((_B, 1), jnp.float32),
            pltpu.VMEM((_B, 1), jnp.int32),
            pltpu.VMEM((_B, 1), jnp.float32),
            pltpu.VMEM((_B, _NUM_FEATURES), jnp.float32),
        ],
        compiler_params=pltpu.CompilerParams(
            dimension_semantics=("arbitrary",)),
    )(lab, inputs, cls_scores, lut, cq)
    return out[0, 0]
